# baseline (speedup 1.0000x reference)
import functools

import jax
import jax.numpy as jnp
from jax import lax
from jax.experimental import pallas as pl
from jax.experimental.pallas import tpu as pltpu
from jax.experimental.pallas import tpu_sc as plsc

VOCAB = 1000000
EMBED = 64
B = 16384
L = 50

CB = 65536
NBLK = 11
SPLIT = NBLK * CB
ZPAD = 1024
TAIL0 = SPLIT + ZPAD
VLO = TAIL0 + 128

NW = 32
HT = 10240
VSC1 = 999936
VSC0 = VSC1 - NW * HT
VHI = NW * HT
CW = HT // 16
NGRP = CW // 16
TAILSHIFT = VSC1 - TAIL0

BPW = B // NW
NCHUNK = BPW // 16
LCH = 10
NGC = L // LCH


def _v_kernel(w_ref, b_ref, t_ref, tail_ref, o_ref):
    scale = 1.0 / L
    w = w_ref[...]
    acc = lax.dot_general(w, t_ref[...], (((1,), (0,)), ((), ())),
                          preferred_element_type=jnp.float32)
    i = pl.program_id(0)
    o_ref[pl.ds(i * CB, CB)] = jnp.reshape(acc * scale + b_ref[0] * scale,
                                           (CB,))

    @pl.when(i == NBLK - 1)
    def _():
        o_ref[pl.ds(SPLIT, ZPAD)] = jnp.zeros((ZPAD,), jnp.float32)
        tacc = lax.dot_general(w, tail_ref[...], (((1,), (0,)), ((), ())),
                               preferred_element_type=jnp.float32)
        o_ref[pl.ds(TAIL0, 128)] = jnp.reshape(
            tacc * scale + b_ref[0] * scale, (128,))


def _compute_v_lo(table_t, fc_w, fc_b):
    return pl.pallas_call(
        _v_kernel,
        grid=(NBLK,),
        in_specs=[
            pl.BlockSpec((1, EMBED), lambda i: (0, 0)),
            pl.BlockSpec(memory_space=pltpu.SMEM),
            pl.BlockSpec((EMBED, CB), lambda i: (0, i)),
            pl.BlockSpec((EMBED, 128), lambda i: (0, VSC1 // 128)),
        ],
        out_specs=pl.BlockSpec((VLO,), lambda i: (0,)),
        out_shape=jax.ShapeDtypeStruct((VLO,), jnp.float32),
    )(fc_w, fc_b, table_t, table_t)


def _vhi_kernel(tt_hbm, w_hbm, b_hbm, vhi_hbm, w_v, b_v, t0, t1, ob, sem):
    nc = 2
    wid = lax.axis_index("s") * nc + lax.axis_index("c")
    col0 = VSC0 + wid * HT
    pltpu.sync_copy(w_hbm, w_v)
    pltpu.sync_copy(b_hbm, b_v)
    scale = 1.0 / L
    bias_vec = b_v[...] * scale
    wv = [w_v[0, pl.ds(j * 16, 16)] for j in range(EMBED // 16)]
    ws = [wv[e // 16][e % 16] for e in range(EMBED)]
    tbufs = (t0, t1)
    first = pltpu.async_copy(tt_hbm.at[:, pl.ds(col0, CW)], t0, sem)
    copies = [first]
    for c in range(16):
        if c < 15:
            copies.append(pltpu.async_copy(
                tt_hbm.at[:, pl.ds(col0 + (c + 1) * CW, CW)],
                tbufs[(c + 1) % 2], sem))
        copies[c].wait()
        tb = tbufs[c % 2]

        def body(g, _, tb=tb):
            acc = jnp.zeros((16,), jnp.float32)
            for e in range(EMBED):
                acc = acc + tb[e, pl.ds(g * 16, 16)] * ws[e]
            ob[pl.ds(g * 16, 16)] = acc * scale + bias_vec
            return 0

        lax.fori_loop(0, NGRP, body, 0)
        pltpu.sync_copy(ob, vhi_hbm.at[pl.ds(wid * HT + c * CW, CW)])
    @pl.when(wid == NW - 1)
    def _():
        def zbody(z, _):
            ob[pl.ds(z * 16, 16)] = jnp.zeros((16,), jnp.float32)
            return 0
        lax.fori_loop(0, ZPAD // 16, zbody, 0)
        pltpu.sync_copy(ob.at[pl.ds(0, ZPAD)], vhi_hbm.at[pl.ds(VHI, ZPAD)])


_vhi_call = functools.partial(
    pl.kernel,
    mesh=plsc.VectorSubcoreMesh(core_axis_name="c", subcore_axis_name="s"),
    out_type=jax.ShapeDtypeStruct((VHI + ZPAD,), jnp.float32),
    scratch_types=[
        pltpu.VMEM((1, EMBED), jnp.float32),
        pltpu.VMEM((16,), jnp.float32),
        pltpu.VMEM((EMBED, CW), jnp.float32),
        pltpu.VMEM((EMBED, CW), jnp.float32),
        pltpu.VMEM((CW,), jnp.float32),
        pltpu.SemaphoreType.DMA,
    ],
)(_vhi_kernel)


def _gather_kernel(vlo_hbm, vhi_hbm, xt_hbm, out_hbm,
                   i0, i1, i2, i3, i4, h0, h1, h2, h3, h4,
                   vlo_v, vhi_v, out_v, sem, gsem):
    nc = 2
    wid = lax.axis_index("s") * nc + lax.axis_index("c")
    idx_bufs = (i0, i1, i2, i3, i4)
    hi_bufs = (h0, h1, h2, h3, h4)
    stage = [[pltpu.async_copy(
        xt_hbm.at[k * LCH + l, pl.ds(wid * BPW, BPW)],
        idx_bufs[k].at[pl.ds(l * BPW, BPW)], sem)
        for l in range(LCH)] for k in range(NGC)]
    gaths = []
    for k in range(NGC):
        for cp in stage[k]:
            cp.wait()

        def tbody(g, _, k=k):
            iv = idx_bufs[k][pl.ds(g * 16, 16)]
            islo = iv < SPLIT
            istail = iv >= VSC1
            idx_bufs[k][pl.ds(g * 16, 16)] = jnp.where(
                islo, iv, jnp.where(istail, iv - TAILSHIFT,
                                    jnp.full((16,), SPLIT, jnp.int32)))
            hi_bufs[k][pl.ds(g * 16, 16)] = jnp.where(
                jnp.logical_or(islo, istail),
                jnp.full((16,), VHI, jnp.int32), iv - VSC0)
            return 0

        lax.fori_loop(0, LCH * BPW // 16, tbody, 0)
        gaths.append(pltpu.async_copy(
            vlo_hbm.at[idx_bufs[k]],
            vlo_v.at[pl.ds(k * LCH * BPW, LCH * BPW)], gsem))
        gaths.append(pltpu.async_copy(
            vhi_hbm.at[hi_bufs[k]],
            vhi_v.at[pl.ds(k * LCH * BPW, LCH * BPW)], gsem))
    for k in range(NGC):
        gaths[2 * k].wait()
        gaths[2 * k + 1].wait()
        for c in range(NCHUNK):
            def body(l, acc, k=k, c=c):
                o = (k * LCH + l) * BPW + c * 16
                return acc + vlo_v[pl.ds(o, 16)] + vhi_v[pl.ds(o, 16)]

            acc = lax.fori_loop(0, LCH, body, jnp.zeros((16,), jnp.float32))
            if k == 0:
                out_v[pl.ds(c * 16, 16)] = acc
            else:
                out_v[pl.ds(c * 16, 16)] = out_v[pl.ds(c * 16, 16)] + acc
    pltpu.sync_copy(out_v, out_hbm.at[pl.ds(wid * BPW, BPW)])


_gather_call = functools.partial(
    pl.kernel,
    mesh=plsc.VectorSubcoreMesh(core_axis_name="c", subcore_axis_name="s"),
    out_type=jax.ShapeDtypeStruct((B,), jnp.float32),
    scratch_types=(
        [pltpu.VMEM((LCH * BPW,), jnp.int32) for _ in range(2 * NGC)] + [
            pltpu.VMEM((L * BPW,), jnp.float32),
            pltpu.VMEM((L * BPW,), jnp.float32),
            pltpu.VMEM((BPW,), jnp.float32),
            pltpu.SemaphoreType.DMA,
            pltpu.SemaphoreType.DMA,
        ]),
)(_gather_kernel)


def kernel(x, table, fc_w, fc_b):
    x = x.astype(jnp.int32)
    tt = table.T
    v_lo = _compute_v_lo(tt, fc_w, fc_b)
    v_hi = _vhi_call(tt, fc_w, jnp.broadcast_to(fc_b, (16,)))
    return _gather_call(v_lo, v_hi, x.T)


# speedup vs baseline: 23.7348x; 23.7348x over previous
import functools

import jax
import jax.numpy as jnp
from jax import lax
from jax.experimental import pallas as pl
from jax.experimental.pallas import tpu as pltpu
from jax.experimental.pallas import tpu_sc as plsc

VOCAB = 1000000
EMBED = 64
B = 16384
L = 50

CB = 65536
NBLK = pl.cdiv(VOCAB, CB)
VP = NBLK * CB

NW = 32
BPW = B // NW
NCHUNK = BPW // 16


def _v_kernel(w_ref, b_ref, t_ref, o_ref):
    scale = 1.0 / L
    acc = lax.dot_general(w_ref[...], t_ref[...], (((1,), (0,)), ((), ())),
                          preferred_element_type=jnp.float32)
    o_ref[...] = jnp.reshape(acc * scale + b_ref[0] * scale, (CB,))


def _compute_v(table_t, fc_w, fc_b):
    return pl.pallas_call(
        _v_kernel,
        grid=(NBLK,),
        in_specs=[
            pl.BlockSpec((1, EMBED), lambda i: (0, 0)),
            pl.BlockSpec(memory_space=pltpu.SMEM),
            pl.BlockSpec((EMBED, CB), lambda i: (0, i)),
        ],
        out_specs=pl.BlockSpec((CB,), lambda i: (i,)),
        out_shape=jax.ShapeDtypeStruct((VP,), jnp.float32),
    )(fc_w, fc_b, table_t)


LCH = 10
NGC = L // LCH


def _gather_kernel(v_hbm, xt_hbm, out_hbm, i0, i1, i2, i3, i4,
                   vals_v, out_v, sem, gsem):
    nc = 2
    wid = lax.axis_index("s") * nc + lax.axis_index("c")
    idx_bufs = (i0, i1, i2, i3, i4)
    stage = [[pltpu.async_copy(
        xt_hbm.at[k * LCH + l, pl.ds(wid * BPW, BPW)],
        idx_bufs[k].at[pl.ds(l * BPW, BPW)], sem)
        for l in range(LCH)] for k in range(NGC)]
    gaths = []
    for k in range(NGC):
        for cp in stage[k]:
            cp.wait()
        gaths.append(pltpu.async_copy(
            v_hbm.at[idx_bufs[k]],
            vals_v.at[pl.ds(k * LCH * BPW, LCH * BPW)], gsem))
    for k in range(NGC):
        gaths[k].wait()
        for c in range(NCHUNK):
            def body(l, acc, k=k, c=c):
                return acc + vals_v[pl.ds((k * LCH + l) * BPW + c * 16, 16)]

            acc = lax.fori_loop(0, LCH, body, jnp.zeros((16,), jnp.float32))
            if k == 0:
                out_v[pl.ds(c * 16, 16)] = acc
            else:
                out_v[pl.ds(c * 16, 16)] = out_v[pl.ds(c * 16, 16)] + acc
    pltpu.sync_copy(out_v, out_hbm.at[pl.ds(wid * BPW, BPW)])


_gather_call = functools.partial(
    pl.kernel,
    mesh=plsc.VectorSubcoreMesh(core_axis_name="c", subcore_axis_name="s"),
    out_type=jax.ShapeDtypeStruct((B,), jnp.float32),
    scratch_types=[
        pltpu.VMEM((LCH * BPW,), jnp.int32),
        pltpu.VMEM((LCH * BPW,), jnp.int32),
        pltpu.VMEM((LCH * BPW,), jnp.int32),
        pltpu.VMEM((LCH * BPW,), jnp.int32),
        pltpu.VMEM((LCH * BPW,), jnp.int32),
        pltpu.VMEM((L * BPW,), jnp.float32),
        pltpu.VMEM((BPW,), jnp.float32),
        pltpu.SemaphoreType.DMA,
        pltpu.SemaphoreType.DMA,
    ],
)(_gather_kernel)


def kernel(x, table, fc_w, fc_b):
    x = x.astype(jnp.int32)
    v = _compute_v(table.T, fc_w, fc_b)
    return _gather_call(v, x.T)
